# baseline (speedup 1.0000x reference)
import jax
import jax.numpy as jnp
from jax.experimental import pallas as pl
from jax.experimental.pallas import tpu as pltpu

EPS = 1e-10

_ROWS = 128
_COLS = 100000
_BAND = 8
_NBANDS = _ROWS // _BAND
_SLOTS = 4


def _band_softmax(l, u):
    t = EPS - jnp.log(u + EPS)
    c = jnp.max(l, axis=-1, keepdims=True)
    p = jnp.exp(l - c) / t
    s = jnp.sum(p, axis=-1, keepdims=True)
    return p * (1.0 / s)


def _gumbel_softmax_kernel(logits_hbm, uniform_hbm, out_hbm,
                           l_buf, u_buf, o_buf, l_sem, u_sem, o_sem):
    def in_copies(band, slot):
        rows = pl.ds(band * _BAND, _BAND)
        return (
            pltpu.make_async_copy(logits_hbm.at[rows, :], l_buf.at[slot], l_sem.at[slot]),
            pltpu.make_async_copy(uniform_hbm.at[rows, :], u_buf.at[slot], u_sem.at[slot]),
        )

    def out_copy(band, slot):
        rows = pl.ds(band * _BAND, _BAND)
        return pltpu.make_async_copy(o_buf.at[slot], out_hbm.at[rows, :], o_sem.at[slot])

    for b in range(_SLOTS):
        for c in in_copies(b, b):
            c.start()

    for b in range(_NBANDS):
        slot = b % _SLOTS
        for c in in_copies(b, slot):
            c.wait()
        if b >= _SLOTS:
            out_copy(b - _SLOTS, slot).wait()
        o_buf[slot] = _band_softmax(l_buf[slot], u_buf[slot])
        out_copy(b, slot).start()
        nb = b + _SLOTS
        if nb < _NBANDS:
            for c in in_copies(nb, slot):
                c.start()

    for b in range(_NBANDS - _SLOTS, _NBANDS):
        out_copy(b, b % _SLOTS).wait()


def kernel(logits, uniform):
    hbm_spec = pl.BlockSpec(memory_space=pltpu.MemorySpace.HBM)
    return pl.pallas_call(
        _gumbel_softmax_kernel,
        in_specs=[hbm_spec, hbm_spec],
        out_specs=hbm_spec,
        out_shape=jax.ShapeDtypeStruct((_ROWS, _COLS), jnp.float32),
        scratch_shapes=[
            pltpu.VMEM((_SLOTS, _BAND, _COLS), jnp.float32),
            pltpu.VMEM((_SLOTS, _BAND, _COLS), jnp.float32),
            pltpu.VMEM((_SLOTS, _BAND, _COLS), jnp.float32),
            pltpu.SemaphoreType.DMA((_SLOTS,)),
            pltpu.SemaphoreType.DMA((_SLOTS,)),
            pltpu.SemaphoreType.DMA((_SLOTS,)),
        ],
    )(logits, uniform)
